# baseline (speedup 1.0000x reference)
import jax
import jax.numpy as jnp
from jax import lax
from jax.experimental import pallas as pl
from jax.experimental.pallas import tpu as pltpu


def _fused_body(eid_ref, x_ref, w1_ref, b1_ref, w2_ref, b2_ref, o_ref):
    x = x_ref[...]
    eid = eid_ref[...]
    outs = []
    for e in range(3):
        w1 = w1_ref[e]
        h = lax.dot_general(x, w1, (((1,), (1,)), ((), ())),
                            preferred_element_type=jnp.float32)
        h = jnp.maximum(h + b1_ref[e][None, :], 0.0)
        o = jnp.sum(h * w2_ref[e][None, :], axis=1, keepdims=True)
        outs.append(o + b2_ref[0, e])
    y = jnp.where(eid == 0, outs[0], jnp.where(eid == 1, outs[1], outs[2]))
    o_ref[...] = y


def kernel(x, atomic_nums, Wc1, bc1, Wc2, bc2, Wh1, bh1, Wh2, bh2,
           Wo1, bo1, Wo2, bo2):
    n, in_dim = x.shape
    hid = Wc1.shape[0]
    tile = 1000
    assert n % tile == 0
    an = atomic_nums.astype(jnp.int32)
    eid = jnp.where(an == 6, 0, jnp.where(an == 1, 1, 2)).astype(jnp.int32)
    eid = eid.reshape(n, 1)
    x = x.astype(jnp.bfloat16)
    w1s = jnp.stack([Wc1, Wh1, Wo1]).astype(jnp.bfloat16)
    b1s = jnp.stack([bc1, bh1, bo1])
    w2s = jnp.stack([Wc2[0], Wh2[0], Wo2[0]])
    b2s = jnp.stack([bc2, bh2, bo2]).reshape(1, 3)

    out = pl.pallas_call(
        _fused_body,
        grid=(n // tile,),
        in_specs=[
            pl.BlockSpec((tile, 1), lambda i: (i, 0)),
            pl.BlockSpec((tile, in_dim), lambda i: (i, 0)),
            pl.BlockSpec((3, hid, in_dim), lambda i: (0, 0, 0)),
            pl.BlockSpec((3, hid), lambda i: (0, 0)),
            pl.BlockSpec((3, hid), lambda i: (0, 0)),
            pl.BlockSpec((1, 3), lambda i: (0, 0)),
        ],
        out_specs=pl.BlockSpec((tile, 1), lambda i: (i, 0)),
        out_shape=jax.ShapeDtypeStruct((n, 1), jnp.float32),
    )(eid, x, w1s, b1s, w2s, b2s)
    return out


# speedup vs baseline: 1.2845x; 1.2845x over previous
import jax
import jax.numpy as jnp
from jax import lax
from jax.experimental import pallas as pl
from jax.experimental.pallas import tpu as pltpu


def _fused_body(eid_ref, x_ref, w1_ref, b1_ref, w2_ref, b2_ref, o_ref):
    x = x_ref[...].astype(jnp.bfloat16)
    eid = eid_ref[0]
    outs = []
    for e in range(3):
        hT = lax.dot_general(w1_ref[e], x, (((1,), (1,)), ((), ())),
                             preferred_element_type=jnp.float32)
        hT = jnp.maximum(hT + b1_ref[e][:, None], 0.0).astype(jnp.bfloat16)
        oT = lax.dot_general(w2_ref[e][None, :], hT, (((1,), (0,)), ((), ())),
                             preferred_element_type=jnp.float32)
        outs.append(oT + b2_ref[0, e])
    y = jnp.where(eid == 0, outs[0], jnp.where(eid == 1, outs[1], outs[2]))
    o_ref[0] = y


def kernel(x, atomic_nums, Wc1, bc1, Wc2, bc2, Wh1, bh1, Wh2, bh2,
           Wo1, bo1, Wo2, bo2):
    n, in_dim = x.shape
    hid = Wc1.shape[0]
    tile = 1000
    assert n % tile == 0
    ntiles = n // tile
    an = atomic_nums.astype(jnp.int32)
    eid = jnp.where(an == 6, 0, jnp.where(an == 1, 1, 2)).astype(jnp.int32)
    eid = eid.reshape(ntiles, 1, tile)
    w1s = jnp.stack([Wc1, Wh1, Wo1]).astype(jnp.bfloat16)
    b1s = jnp.stack([bc1, bh1, bo1])
    w2s = jnp.stack([Wc2[0], Wh2[0], Wo2[0]]).astype(jnp.bfloat16)
    b2s = jnp.stack([bc2, bh2, bo2]).reshape(1, 3)

    out = pl.pallas_call(
        _fused_body,
        grid=(ntiles,),
        in_specs=[
            pl.BlockSpec((1, 1, tile), lambda i: (i, 0, 0)),
            pl.BlockSpec((tile, in_dim), lambda i: (i, 0)),
            pl.BlockSpec((3, hid, in_dim), lambda i: (0, 0, 0)),
            pl.BlockSpec((3, hid), lambda i: (0, 0)),
            pl.BlockSpec((3, hid), lambda i: (0, 0)),
            pl.BlockSpec((1, 3), lambda i: (0, 0)),
        ],
        out_specs=pl.BlockSpec((1, 1, tile), lambda i: (i, 0, 0)),
        out_shape=jax.ShapeDtypeStruct((ntiles, 1, tile), jnp.float32),
    )(eid, x, w1s, b1s, w2s, b2s)
    return out.reshape(n, 1)
